# baseline (device time: 215510 ns/iter reference)
import functools

import numpy as np
import jax
import jax.numpy as jnp
from jax import lax
from jax.experimental import pallas as pl
from jax.experimental.pallas import tpu as pltpu

N_DEV = 8
B, SQ, D = 2, 512, 1024
HL, DH = 8, 128
SCALE = 0.08838834764831843


def _rope_consts():
    inv = 1.0 / (10000.0 ** (np.arange(0, DH, 2) / DH))
    pos = np.arange(SQ)[:, None] * inv[None, :]
    cos = np.repeat(np.cos(pos), 2, axis=-1).astype(np.float32)
    sin = np.repeat(np.sin(pos), 2, axis=-1).astype(np.float32)
    R = np.zeros((DH, DH), dtype=np.float32)
    idx = np.arange(0, DH, 2)
    R[idx + 1, idx] = -1.0
    R[idx, idx + 1] = 1.0
    return cos, sin, R


def _body(x_ref, wq_ref, wk_ref, wv_ref, wo_ref, cos_ref, sin_ref, r_ref,
          out_ref, ctx_ref, comm_ref, send_sems, recv_sems, credit_sem):
    my = lax.axis_index("i")
    left = lax.rem(my + (N_DEV - 1), N_DEV)
    right = lax.rem(my + 1, N_DEV)

    barrier_sem = pltpu.get_barrier_semaphore()
    for nbr in (left, right):
        pl.semaphore_signal(barrier_sem, inc=1, device_id=(nbr,),
                            device_id_type=pl.DeviceIdType.MESH)
    pl.semaphore_wait(barrier_sem, 2)

    xb = x_ref[...].astype(jnp.bfloat16)
    f32 = jnp.float32
    q_all = jnp.dot(xb, wq_ref[...].astype(jnp.bfloat16), preferred_element_type=f32)
    k_all = jnp.dot(xb, wk_ref[...].astype(jnp.bfloat16), preferred_element_type=f32)
    v_all = jnp.dot(xb, wv_ref[...].astype(jnp.bfloat16), preferred_element_type=f32)

    cos = cos_ref[...]
    sin = sin_ref[...]
    Rb = r_ref[...].astype(jnp.bfloat16)

    for b in range(B):
        rows = slice(b * SQ, (b + 1) * SQ)
        for h in range(HL):
            cols = slice(h * DH, (h + 1) * DH)
            q = q_all[rows, cols]
            k = k_all[rows, cols]
            q = q * cos + jnp.dot(q.astype(jnp.bfloat16), Rb,
                                  preferred_element_type=f32) * sin
            k = k * cos + jnp.dot(k.astype(jnp.bfloat16), Rb,
                                  preferred_element_type=f32) * sin
            s = lax.dot_general(
                q.astype(jnp.bfloat16), k.astype(jnp.bfloat16),
                (((1,), (1,)), ((), ())), preferred_element_type=f32) * SCALE
            m = jnp.max(s, axis=-1, keepdims=True)
            e = jnp.exp(s - m)
            den = jnp.sum(e, axis=-1, keepdims=True)
            w = (e / den).astype(jnp.bfloat16)
            v = v_all[rows, cols].astype(jnp.bfloat16)
            ctx_ref[rows, cols] = jnp.dot(
                w, v, preferred_element_type=f32).astype(jnp.bfloat16)

    partial = jnp.dot(ctx_ref[...], wo_ref[...].astype(jnp.bfloat16),
                      preferred_element_type=f32)

    comm_ref[0] = partial.astype(jnp.bfloat16)
    acc = partial
    for h in range(N_DEV - 1):
        s_slot = h % 2
        r_slot = (h + 1) % 2
        if h >= 1:
            pl.semaphore_wait(credit_sem, 1)
        rdma = pltpu.make_async_remote_copy(
            src_ref=comm_ref.at[s_slot],
            dst_ref=comm_ref.at[r_slot],
            send_sem=send_sems.at[s_slot],
            recv_sem=recv_sems.at[r_slot],
            device_id=(right,),
            device_id_type=pl.DeviceIdType.MESH,
        )
        rdma.start()
        rdma.wait_send()
        if h <= N_DEV - 3:
            pl.semaphore_signal(credit_sem, inc=1, device_id=(left,),
                                device_id_type=pl.DeviceIdType.MESH)
        rdma.wait_recv()
        acc = acc + comm_ref[r_slot].astype(f32)

    out_ref[...] = acc


def kernel(x, Wq, Wk, Wv, Wo):
    cos, sin, R = _rope_consts()
    xf = x.reshape(B * SQ, D)

    out = pl.pallas_call(
        _body,
        out_shape=jax.ShapeDtypeStruct((B * SQ, D), jnp.float32),
        in_specs=[pl.BlockSpec(memory_space=pltpu.VMEM)] * 8,
        out_specs=pl.BlockSpec(memory_space=pltpu.VMEM),
        scratch_shapes=[
            pltpu.VMEM((B * SQ, HL * DH), jnp.bfloat16),
            pltpu.VMEM((2, B * SQ, D), jnp.bfloat16),
            pltpu.SemaphoreType.DMA((2,)),
            pltpu.SemaphoreType.DMA((2,)),
            pltpu.SemaphoreType.REGULAR,
        ],
        compiler_params=pltpu.CompilerParams(collective_id=0),
    )(xf, Wq, Wk, Wv, Wo, jnp.asarray(cos), jnp.asarray(sin), jnp.asarray(R))
    return out.reshape(B, SQ, D)


# device time: 73065 ns/iter; 2.9496x vs baseline; 2.9496x over previous
import functools

import numpy as np
import jax
import jax.numpy as jnp
from jax import lax
from jax.experimental import pallas as pl
from jax.experimental.pallas import tpu as pltpu

N_DEV = 8
B, SQ, D = 2, 512, 1024
HL, DH = 8, 128
SCALE = 0.08838834764831843


def _rope_consts():
    inv = 1.0 / (10000.0 ** (np.arange(0, DH, 2) / DH))
    pos = np.arange(SQ)[:, None] * inv[None, :]
    cos = np.repeat(np.cos(pos), 2, axis=-1).astype(np.float32)
    sin = np.repeat(np.sin(pos), 2, axis=-1).astype(np.float32)
    R = np.zeros((DH, DH), dtype=np.float32)
    idx = np.arange(0, DH, 2)
    R[idx + 1, idx] = -1.0
    R[idx, idx + 1] = 1.0
    return cos, sin, R


_MASKS = (1, 3, 4)


def _body(x_ref, wq_ref, wk_ref, wv_ref, wo_ref, cos_ref, sin_ref, r_ref,
          out_ref, ctx_ref, send_pool, recv_pool, send_sems, recv_sems):
    my = lax.axis_index("i")

    barrier_sem = pltpu.get_barrier_semaphore()
    for mask in _MASKS:
        pl.semaphore_signal(barrier_sem, inc=1,
                            device_id=(jnp.bitwise_xor(my, mask),),
                            device_id_type=pl.DeviceIdType.MESH)
    pl.semaphore_wait(barrier_sem, 3)

    xb = x_ref[...].astype(jnp.bfloat16)
    f32 = jnp.float32
    q_all = jnp.dot(xb, wq_ref[...].astype(jnp.bfloat16), preferred_element_type=f32)
    k_all = jnp.dot(xb, wk_ref[...].astype(jnp.bfloat16), preferred_element_type=f32)
    v_all = jnp.dot(xb, wv_ref[...].astype(jnp.bfloat16), preferred_element_type=f32)

    cos = cos_ref[...]
    sin = sin_ref[...]
    Rb = r_ref[...].astype(jnp.bfloat16)

    for b in range(B):
        rows = slice(b * SQ, (b + 1) * SQ)
        for h in range(HL):
            cols = slice(h * DH, (h + 1) * DH)
            q = q_all[rows, cols]
            k = k_all[rows, cols]
            q = q * cos + jnp.dot(q.astype(jnp.bfloat16), Rb,
                                  preferred_element_type=f32) * sin
            k = k * cos + jnp.dot(k.astype(jnp.bfloat16), Rb,
                                  preferred_element_type=f32) * sin
            s = lax.dot_general(
                q.astype(jnp.bfloat16), k.astype(jnp.bfloat16),
                (((1,), (1,)), ((), ())), preferred_element_type=f32) * SCALE
            m = jnp.max(s, axis=-1, keepdims=True)
            e = jnp.exp(s - m)
            den = jnp.sum(e, axis=-1, keepdims=True)
            w = (e / den).astype(jnp.bfloat16)
            v = v_all[rows, cols].astype(jnp.bfloat16)
            ctx_ref[rows, cols] = jnp.dot(
                w, v, preferred_element_type=f32).astype(jnp.bfloat16)

    out_ref[...] = jnp.dot(ctx_ref[...], wo_ref[...].astype(jnp.bfloat16),
                           preferred_element_type=f32)

    bit = {
        1: jnp.bitwise_xor(my & 1, (my >> 1) & 1),
        3: (my >> 1) & 1,
        4: (my >> 2) & 1,
    }
    halves = [
        {"start": 0, "rows": 512, "rs": (1, 3, 4), "ag": (4, 3, 1),
         "pool_base": 0},
        {"start": 512, "rows": 512, "rs": (3, 4, 1), "ag": (1, 4, 3),
         "pool_base": 896},
    ]
    rs_off = (0, 256, 384)
    rs_rows = (256, 128, 64)
    ag_off = (448, 512, 640)
    ag_rows = (64, 128, 256)

    for s in range(6):
        pend = []
        for idx, H in enumerate(halves):
            sem = 2 * s + idx
            if s < 3:
                mask = H["rs"][s]
                n = rs_rows[s]
                po = H["pool_base"] + rs_off[s]
                hi = bit[mask]
                keep = H["start"] + hi * n
                send_off = H["start"] + (1 - hi) * n
                send_pool[pl.ds(po, n), :] = (
                    out_ref[pl.ds(send_off, n), :].astype(jnp.bfloat16))
                rdma = pltpu.make_async_remote_copy(
                    src_ref=send_pool.at[pl.ds(po, n)],
                    dst_ref=recv_pool.at[pl.ds(po, n)],
                    send_sem=send_sems.at[sem],
                    recv_sem=recv_sems.at[sem],
                    device_id=(jnp.bitwise_xor(my, mask),),
                    device_id_type=pl.DeviceIdType.MESH,
                )
                rdma.start()
                pend.append((rdma, "rs", keep, n, po))
                H["start"] = keep
            else:
                t = s - 3
                mask = H["ag"][t]
                n = ag_rows[t]
                po = H["pool_base"] + ag_off[t]
                hi = bit[mask]
                send_pool[pl.ds(po, n), :] = (
                    out_ref[pl.ds(H["start"], n), :].astype(jnp.bfloat16))
                rdma = pltpu.make_async_remote_copy(
                    src_ref=send_pool.at[pl.ds(po, n)],
                    dst_ref=recv_pool.at[pl.ds(po, n)],
                    send_sem=send_sems.at[sem],
                    recv_sem=recv_sems.at[sem],
                    device_id=(jnp.bitwise_xor(my, mask),),
                    device_id_type=pl.DeviceIdType.MESH,
                )
                rdma.start()
                other = H["start"] + (1 - 2 * hi) * n
                pend.append((rdma, "ag", other, n, po))
                H["start"] = H["start"] - hi * n
        for rdma, kind, off, n, po in pend:
            rdma.wait()
        for rdma, kind, off, n, po in pend:
            if kind == "rs":
                out_ref[pl.ds(off, n), :] = (
                    out_ref[pl.ds(off, n), :]
                    + recv_pool[pl.ds(po, n), :].astype(f32))
            else:
                out_ref[pl.ds(off, n), :] = (
                    recv_pool[pl.ds(po, n), :].astype(f32))


def kernel(x, Wq, Wk, Wv, Wo):
    cos, sin, R = _rope_consts()
    xf = x.reshape(B * SQ, D)

    out = pl.pallas_call(
        _body,
        out_shape=jax.ShapeDtypeStruct((B * SQ, D), jnp.float32),
        in_specs=[pl.BlockSpec(memory_space=pltpu.VMEM)] * 8,
        out_specs=pl.BlockSpec(memory_space=pltpu.VMEM),
        scratch_shapes=[
            pltpu.VMEM((B * SQ, HL * DH), jnp.bfloat16),
            pltpu.VMEM((1792, D), jnp.bfloat16),
            pltpu.VMEM((1792, D), jnp.bfloat16),
            pltpu.SemaphoreType.DMA((12,)),
            pltpu.SemaphoreType.DMA((12,)),
        ],
        compiler_params=pltpu.CompilerParams(collective_id=0),
    )(xf, Wq, Wk, Wv, Wo, jnp.asarray(cos), jnp.asarray(sin), jnp.asarray(R))
    return out.reshape(B, SQ, D)
